# baseline (speedup 1.0000x reference)
import functools

import jax
import jax.numpy as jnp
from jax import lax
from jax.experimental import pallas as pl
from jax.experimental.pallas import tpu as pltpu
from jax.experimental.pallas import tpu_sc as plsc

TOK = 16384
E = 64
NC = 2
NS = 16
L = 16
NW = NC * NS
TPW = TOK // NW
TCH = 256
NCH = TPW // TCH
NEG = -1e30


def _sc_body(xt_hbm, w_hbm, i_hbm,
             x0, x1, w0, w1, j0, j1,
             sin0, sin1, sw0, sw1, si0, si1):
    wid = lax.axis_index("s") * NC + lax.axis_index("c")
    base = wid * TPW

    xb, wb, jb = [x0, x1], [w0, w1], [j0, j1]
    sin, sw, si = [sin0, sin1], [sw0, sw1], [si0, si1]

    def start_in(c):
        return pltpu.async_copy(
            xt_hbm.at[:, pl.ds(base + c * TCH, TCH)], xb[c % 2], sin[c % 2])

    def compute(c):
        p = c % 2

        def group(g, carry):
            m1 = jnp.full((L,), NEG, jnp.float32)
            m2 = jnp.full((L,), NEG, jnp.float32)
            i1 = jnp.zeros((L,), jnp.int32)
            i2 = jnp.zeros((L,), jnp.int32)
            s = jnp.zeros((L,), jnp.float32)
            e0 = jnp.full((L,), -1, jnp.int32)

            def estep(e, st):
                m1, m2, i1, i2, s, ev = st
                v = xb[p][e, pl.ds(g * L, L)]
                ev = ev + 1
                gt1 = v > m1
                gt2 = v > m2
                i2 = jnp.where(gt1, i1, jnp.where(gt2, ev, i2))
                m2 = jnp.maximum(m2, jnp.minimum(m1, v))
                i1 = jnp.where(gt1, ev, i1)
                m1 = jnp.maximum(m1, v)
                s = s + jnp.exp(v)
                return (m1, m2, i1, i2, s, ev)

            m1, m2, i1, i2, s, _ = lax.fori_loop(
                0, E, estep, (m1, m2, i1, i2, s, e0), unroll=8)
            inv = 1.0 / s
            wb[p][0, pl.ds(g * L, L)] = jnp.exp(m1) * inv
            wb[p][1, pl.ds(g * L, L)] = jnp.exp(m2) * inv
            jb[p][0, pl.ds(g * L, L)] = i1
            jb[p][1, pl.ds(g * L, L)] = i2
            return carry

        lax.fori_loop(0, TCH // L, group, 0)

    def start_out(c):
        p = c % 2
        hw = pltpu.async_copy(
            wb[p], w_hbm.at[:, pl.ds(base + c * TCH, TCH)], sw[p])
        hi = pltpu.async_copy(
            jb[p], i_hbm.at[:, pl.ds(base + c * TCH, TCH)], si[p])
        return hw, hi

    in_h = {0: start_in(0)}
    out_h = {}
    for c in range(NCH):
        if c + 1 < NCH:
            in_h[c + 1] = start_in(c + 1)
        in_h[c].wait()
        if c >= 2:
            for h in out_h[c - 2]:
                h.wait()
        compute(c)
        out_h[c] = start_out(c)
    for c in (NCH - 2, NCH - 1):
        for h in out_h[c]:
            h.wait()


_topk_call = functools.partial(
    pl.kernel,
    out_type=(jax.ShapeDtypeStruct((2, TOK), jnp.float32),
              jax.ShapeDtypeStruct((2, TOK), jnp.int32)),
    mesh=plsc.VectorSubcoreMesh(core_axis_name="c", subcore_axis_name="s",
                                num_cores=NC, num_subcores=NS),
    scratch_types=[
        pltpu.VMEM((E, TCH), jnp.float32),
        pltpu.VMEM((E, TCH), jnp.float32),
        pltpu.VMEM((2, TCH), jnp.float32),
        pltpu.VMEM((2, TCH), jnp.float32),
        pltpu.VMEM((2, TCH), jnp.int32),
        pltpu.VMEM((2, TCH), jnp.int32),
        pltpu.SemaphoreType.DMA,
        pltpu.SemaphoreType.DMA,
        pltpu.SemaphoreType.DMA,
        pltpu.SemaphoreType.DMA,
        pltpu.SemaphoreType.DMA,
        pltpu.SemaphoreType.DMA,
    ],
    compiler_params=pltpu.CompilerParams(needs_layout_passes=False,
                                         use_tc_tiling_on_sc=True),
)(_sc_body)


def kernel(gating_output, topk):
    del topk
    wt, it = _topk_call(gating_output.T)
    return (wt.T, it.T)


# speedup vs baseline: 1.0272x; 1.0272x over previous
import functools

import jax
import jax.numpy as jnp
from jax import lax
from jax.experimental import pallas as pl
from jax.experimental.pallas import tpu as pltpu
from jax.experimental.pallas import tpu_sc as plsc

TOK = 16384
E = 64
NC = 2
NS = 16
L = 16
NW = NC * NS
TPW = TOK // NW
TCH = 128
NCH = TPW // TCH
NEG = -1e30


def _sc_body(xt_hbm, w_hbm, i_hbm,
             x0, x1, w0, w1, j0, j1,
             sin0, sin1, sw0, sw1, si0, si1):
    wid = lax.axis_index("s") * NC + lax.axis_index("c")
    base = wid * TPW

    xb, wb, jb = [x0, x1], [w0, w1], [j0, j1]
    sin, sw, si = [sin0, sin1], [sw0, sw1], [si0, si1]

    def start_in(c):
        return pltpu.async_copy(
            xt_hbm.at[:, pl.ds(base + c * TCH, TCH)], xb[c % 2], sin[c % 2])

    def compute(c):
        p = c % 2

        def group(g, carry):
            m1 = jnp.full((L,), NEG, jnp.float32)
            m2 = jnp.full((L,), NEG, jnp.float32)
            i1 = jnp.zeros((L,), jnp.int32)
            i2 = jnp.zeros((L,), jnp.int32)
            s = jnp.zeros((L,), jnp.float32)
            e0 = jnp.full((L,), -1, jnp.int32)

            def estep(e, st):
                m1, m2, i1, i2, s, ev = st
                v = xb[p][e, pl.ds(g * L, L)]
                ev = ev + 1
                gt1 = v > m1
                gt2 = v > m2
                i2 = jnp.where(gt1, i1, jnp.where(gt2, ev, i2))
                m2 = jnp.maximum(m2, jnp.minimum(m1, v))
                i1 = jnp.where(gt1, ev, i1)
                m1 = jnp.maximum(m1, v)
                s = s + jnp.exp(v)
                return (m1, m2, i1, i2, s, ev)

            m1, m2, i1, i2, s, _ = lax.fori_loop(
                0, E, estep, (m1, m2, i1, i2, s, e0), unroll=4)
            inv = 1.0 / s
            wb[p][0, pl.ds(g * L, L)] = jnp.exp(m1) * inv
            wb[p][1, pl.ds(g * L, L)] = jnp.exp(m2) * inv
            jb[p][0, pl.ds(g * L, L)] = i1
            jb[p][1, pl.ds(g * L, L)] = i2
            return carry

        lax.fori_loop(0, TCH // L, group, 0)

    def start_out(c):
        p = c % 2
        hw = pltpu.async_copy(
            wb[p], w_hbm.at[:, pl.ds(base + c * TCH, TCH)], sw[p])
        hi = pltpu.async_copy(
            jb[p], i_hbm.at[:, pl.ds(base + c * TCH, TCH)], si[p])
        return hw, hi

    in_h = {0: start_in(0)}
    out_h = {}
    for c in range(NCH):
        if c + 1 < NCH:
            in_h[c + 1] = start_in(c + 1)
        in_h[c].wait()
        if c >= 2:
            for h in out_h[c - 2]:
                h.wait()
        compute(c)
        out_h[c] = start_out(c)
    for c in (NCH - 2, NCH - 1):
        for h in out_h[c]:
            h.wait()


_topk_call = functools.partial(
    pl.kernel,
    out_type=(jax.ShapeDtypeStruct((2, TOK), jnp.float32),
              jax.ShapeDtypeStruct((2, TOK), jnp.int32)),
    mesh=plsc.VectorSubcoreMesh(core_axis_name="c", subcore_axis_name="s",
                                num_cores=NC, num_subcores=NS),
    scratch_types=[
        pltpu.VMEM((E, TCH), jnp.float32),
        pltpu.VMEM((E, TCH), jnp.float32),
        pltpu.VMEM((2, TCH), jnp.float32),
        pltpu.VMEM((2, TCH), jnp.float32),
        pltpu.VMEM((2, TCH), jnp.int32),
        pltpu.VMEM((2, TCH), jnp.int32),
        pltpu.SemaphoreType.DMA,
        pltpu.SemaphoreType.DMA,
        pltpu.SemaphoreType.DMA,
        pltpu.SemaphoreType.DMA,
        pltpu.SemaphoreType.DMA,
        pltpu.SemaphoreType.DMA,
    ],
    compiler_params=pltpu.CompilerParams(needs_layout_passes=False,
                                         use_tc_tiling_on_sc=True),
)(_sc_body)


def kernel(gating_output, topk):
    del topk
    wt, it = _topk_call(gating_output.T)
    return (wt.T, it.T)
